# Initial kernel scaffold; baseline (speedup 1.0000x reference)
import jax
import jax.numpy as jnp
from jax.experimental import pallas as pl


def kernel(x, params, edge_index, batch):
    raise NotImplementedError("write your pallas kernel here")


# speedup vs baseline: 1.7535x; 1.7535x over previous
import functools

import jax
import jax.numpy as jnp
from jax import lax
from jax.experimental import pallas as pl
from jax.experimental.pallas import tpu as pltpu
from jax.experimental.pallas import tpu_sc as plsc

_N = 10000
_E = 320000
_D = 128
_G = 64

_NC = 2
_NS = 16
_NW = _NC * _NS

_CHUNK = 128
_EPT = -(-_E // _NW)
_CHUNKS = -(-_EPT // _CHUNK)
_E_PAD = _NW * _CHUNKS * _CHUNK
_AGG_ROWS = 10112
_ZROWS = _AGG_ROWS // _NS
_OROWS = 624
_OTAIL = _N - _NS * _OROWS


_SEGBUF = 128
_EPT = _CHUNKS * _CHUNK
_NGRP = _EPT // 16


def _agg_body(h_hbm, srcs_hbm, dsts_hbm, zeros_hbm, out_hbm,
              src_v, dst_v, rows_v, acc_v, seg_rows_v, seg_idx_v, agg_sh, sem):
    c = lax.axis_index("c")
    s = lax.axis_index("s")
    wid = c * _NS + s

    pltpu.sync_copy(srcs_hbm.at[wid], src_v)
    pltpu.sync_copy(dsts_hbm.at[wid], dst_v)

    pltpu.sync_copy(zeros_hbm, agg_sh.at[pl.ds(s * _ZROWS, _ZROWS)])
    for sub in range(_D // 16):
        acc_v[0, pl.ds(sub * 16, 16)] = jnp.zeros((16,), jnp.float32)
    plsc.subcore_barrier()

    lane0 = lax.broadcasted_iota(jnp.int32, (16,), 0) == 0

    def reset_idx():
        for g16 in range(_SEGBUF // 16):
            seg_idx_v[pl.ds(g16 * 16, 16)] = jnp.full((16,), _N, jnp.int32)

    def stage(cnt, dprev):
        for sub in range(_D // 16):
            sl = pl.ds(sub * 16, 16)
            seg_rows_v[cnt, sl] = acc_v[0, sl]
            acc_v[0, sl] = jnp.zeros((16,), jnp.float32)
        grp = (cnt // 16) * 16
        lane = cnt - grp
        m = lax.broadcasted_iota(jnp.int32, (16,), 0) == lane
        cur = seg_idx_v[pl.ds(grp, 16)]
        seg_idx_v[pl.ds(grp, 16)] = jnp.where(m, dprev, cur)
        cnt2 = cnt + 1

        @pl.when(cnt2 == _SEGBUF)
        def _scatter():
            pltpu.sync_copy(seg_rows_v, agg_sh.at[seg_idx_v], add=True)
            reset_idx()

        return lax.select(cnt2 == _SEGBUF, 0, cnt2)

    reset_idx()

    def group_body(j, carry):
        dprev, cnt = carry
        pltpu.async_copy(
            h_hbm.at[src_v.at[pl.ds(j * 16, 16)]], rows_v, sem).wait()
        dvec = dst_v[pl.ds(j * 16, 16)]
        for e16 in range(16):
            dcur = dvec[e16]
            cnt = lax.cond(dcur != dprev,
                           lambda c, d=dprev: stage(c, d),
                           lambda c: c, cnt)
            for sub in range(_D // 16):
                sl = pl.ds(sub * 16, 16)
                acc_v[0, sl] = acc_v[0, sl] + rows_v[e16, sl]
            dprev = dcur
        return (dprev, cnt)

    d0vec = dst_v[pl.ds(0, 16)]
    dlast, cnt = lax.fori_loop(0, _NGRP, group_body, (d0vec[0], 0))
    stage(cnt, dlast)
    pltpu.sync_copy(seg_rows_v, agg_sh.at[seg_idx_v], add=True)
    plsc.subcore_barrier()

    pltpu.sync_copy(agg_sh.at[pl.ds(s * _OROWS, _OROWS)],
                    out_hbm.at[c, pl.ds(s * _OROWS, _OROWS)])

    @pl.when(s == 0)
    def _tail():
        pltpu.sync_copy(agg_sh.at[pl.ds(_NS * _OROWS, _OTAIL)],
                        out_hbm.at[c, pl.ds(_NS * _OROWS, _OTAIL)])


@functools.cache
def _agg_call():
    return pl.kernel(
        _agg_body,
        mesh=plsc.VectorSubcoreMesh(core_axis_name="c", subcore_axis_name="s"),
        out_type=jax.ShapeDtypeStruct((_NC, _N, _D), jnp.float32),
        scratch_types=[
            pltpu.VMEM((_EPT,), jnp.int32),
            pltpu.VMEM((_EPT,), jnp.int32),
            pltpu.VMEM((16, _D), jnp.float32),
            pltpu.VMEM((1, _D), jnp.float32),
            pltpu.VMEM((_SEGBUF, _D), jnp.float32),
            pltpu.VMEM((_SEGBUF,), jnp.int32),
            pltpu.VMEM_SHARED((_AGG_ROWS, _D), jnp.float32),
            pltpu.SemaphoreType.DMA,
        ],
    )


def _bn_relu(h, g, b):
    mu = jnp.mean(h, axis=0, keepdims=True)
    var = jnp.mean((h - mu) ** 2, axis=0, keepdims=True)
    return jnp.maximum((h - mu) / jnp.sqrt(var + 1e-5) * g + b, 0.0)


def _dot(a, b):
    return jnp.dot(a, b, preferred_element_type=jnp.float32)


def _mm_body(a_ref, w_ref, b_ref, o_ref):
    o_ref[...] = _dot(a_ref[...], w_ref[...]) + b_ref[...]


def _mm_in_body(h_ref, p_ref, eps_ref, w_ref, b_ref, o_ref):
    hin = eps_ref[...] * h_ref[...] + p_ref[0] + p_ref[1]
    o_ref[...] = _dot(hin, w_ref[...]) + b_ref[...]


def _norm_body(t_ref, g_ref, b_ref, o_ref):
    o_ref[...] = _bn_relu(t_ref[...], g_ref[...], b_ref[...])


def _norm_res_body(t_ref, g_ref, b_ref, h_ref, o_ref):
    o_ref[...] = h_ref[...] + _bn_relu(t_ref[...], g_ref[...], b_ref[...])


def _readout_body(h_ref, bat_ref, w1_ref, b1_ref, g_ref, bb_ref,
                  w2_ref, b2_ref, w3_ref, b3_ref, o_ref):
    ids = lax.broadcasted_iota(jnp.int32, (_G, _N), 0)
    oh = (ids == bat_ref[...]).astype(jnp.float32)
    sums = jnp.dot(oh, h_ref[...], preferred_element_type=jnp.float32,
                   precision=lax.Precision.HIGHEST)
    counts = jnp.sum(oh, axis=1, keepdims=True)
    means = sums / jnp.maximum(counts, 1.0)
    t = jnp.concatenate([means, sums], axis=1)
    t = _dot(t, w1_ref[...]) + b1_ref[...]
    t = _bn_relu(t, g_ref[...], bb_ref[...])
    t = jnp.maximum(_dot(t, w2_ref[...]) + b2_ref[...], 0.0)
    o_ref[...] = _dot(t, w3_ref[...]) + b3_ref[...]


def _tc_call(body, out_shape):
    return pl.pallas_call(
        body, out_shape=jax.ShapeDtypeStruct(out_shape, jnp.float32))


def kernel(x, params, edge_index, batch):
    p = params
    order = jnp.argsort(edge_index[1], stable=True)
    src = edge_index[0][order]
    dst = edge_index[1][order]
    pad = _E_PAD - _E
    src_p = jnp.concatenate([src, jnp.zeros((pad,), jnp.int32)])
    dst_p = jnp.concatenate([dst, jnp.full((pad,), _N, jnp.int32)])
    srcs = src_p.reshape(_NW, _EPT)
    dsts = dst_p.reshape(_NW, _EPT)
    zeros = jnp.zeros((_ZROWS, _D), jnp.float32)

    r1 = lambda a: a.reshape(1, -1)

    t = _tc_call(_mm_body, (_N, _D))(x, p['proj_W'], r1(p['proj_b']))
    h = _tc_call(_norm_body, (_N, _D))(
        t, r1(p['proj_bn_g']), r1(p['proj_bn_b']))

    for l in range(4):
        part = _agg_call()(h, srcs, dsts, zeros)
        eps_row = jnp.full((1, _D), 1.0, jnp.float32) + p[f'eps{l}']
        t1 = _tc_call(_mm_in_body, (_N, 2 * _D))(
            h, part, eps_row, p[f'l{l}_W1'], r1(p[f'l{l}_b1']))
        a1 = _tc_call(_norm_body, (_N, 2 * _D))(
            t1, r1(p[f'l{l}_bn1_g']), r1(p[f'l{l}_bn1_b']))
        t2 = _tc_call(_mm_body, (_N, _D))(a1, p[f'l{l}_W2'], r1(p[f'l{l}_b2']))
        h = _tc_call(_norm_res_body, (_N, _D))(
            t2, r1(p[f'l{l}_bn2_g']), r1(p[f'l{l}_bn2_b']), h)

    out = _tc_call(_readout_body, (_G, 1))(
        h, batch.reshape(1, _N),
        p['r_W1'], r1(p['r_b1']), r1(p['r_bn_g']), r1(p['r_bn_b']),
        p['r_W2'], r1(p['r_b2']), p['r_W3'], r1(p['r_b3']))
    return out.reshape(_G)
